# baseline (speedup 1.0000x reference)
import functools

import jax
import jax.numpy as jnp
from jax import lax
from jax.experimental import pallas as pl
from jax.experimental.pallas import tpu as pltpu
from jax.experimental.pallas import tpu_sc as plsc

N0, N1, N2 = 2048, 1024, 512
G_DIM = 128
EMB_CHAN = 16
PATCH = 16
P_DIM = EMB_CHAN * PATCH * PATCH
FUSED = 16
D_MODEL = 2 * G_DIM
TF_HEADS = 4
HEAD_DIM = D_MODEL // TF_HEADS
GAT_HEADS = 2
E0, E1 = 32768, 16384
NEG = -1e30

RB = 256

NC, NS, LANES = 2, 16, 16
NW = NC * NS


def _f32dot(a, b):
    return jnp.dot(a, b, preferred_element_type=jnp.float32)


def _down_body(gr, pr, wr, br, gt, pt, outr):
    i = pl.program_id(0)
    pp = _f32dot(pr[...], wr[...]) + br[...]
    gg = gr[...]
    masked = i < (N2 // RB)
    gg = jnp.where(masked, jnp.broadcast_to(gt[...], gg.shape), gg)
    pp = jnp.where(masked, jnp.broadcast_to(pt[...], pp.shape), pp)
    outr[...] = jnp.concatenate([gg, pp], axis=1)


def _down(g, p_flat, w, b, gtok, ptok):
    return pl.pallas_call(
        _down_body,
        grid=(N0 // RB,),
        in_specs=[
            pl.BlockSpec((RB, G_DIM), lambda i: (i, 0)),
            pl.BlockSpec((RB, P_DIM), lambda i: (i, 0)),
            pl.BlockSpec((P_DIM, G_DIM), lambda i: (0, 0)),
            pl.BlockSpec((1, G_DIM), lambda i: (0, 0)),
            pl.BlockSpec((1, G_DIM), lambda i: (0, 0)),
            pl.BlockSpec((1, G_DIM), lambda i: (0, 0)),
        ],
        out_specs=pl.BlockSpec((RB, D_MODEL), lambda i: (i, 0)),
        out_shape=jax.ShapeDtypeStruct((N0, D_MODEL), jnp.float32),
    )(g, p_flat, w, b, gtok, ptok)


def _qkv_body(xr, wr, br, outr):
    outr[0] = _f32dot(xr[...], wr[0]) + br[0]


def _qkv(x, w3, b3):
    n = x.shape[0]
    nh = 3 * TF_HEADS
    return pl.pallas_call(
        _qkv_body,
        grid=(nh,),
        in_specs=[
            pl.BlockSpec((n, D_MODEL), lambda j: (0, 0)),
            pl.BlockSpec((1, D_MODEL, HEAD_DIM), lambda j: (j, 0, 0)),
            pl.BlockSpec((1, 1, HEAD_DIM), lambda j: (j, 0, 0)),
        ],
        out_specs=pl.BlockSpec((1, n, HEAD_DIM), lambda j: (j, 0, 0)),
        out_shape=jax.ShapeDtypeStruct((nh, n, HEAD_DIM), jnp.float32),
    )(x, w3, b3)


def _attn_body(qr, kr, vr, outr, *, nsrc):
    i = pl.program_id(1)
    s = lax.dot_general(qr[0], kr[0], (((1,), (1,)), ((), ())),
                        preferred_element_type=jnp.float32) * (1.0 / 8.0)
    rows = i * RB + lax.broadcasted_iota(jnp.int32, (RB, nsrc), 0)
    cols = lax.broadcasted_iota(jnp.int32, (RB, nsrc), 1)
    s = jnp.where((rows < N2) & (cols < N2), NEG, s)
    m = jnp.max(s, axis=1, keepdims=True)
    e = jnp.exp(s - m)
    a = e / jnp.sum(e, axis=1, keepdims=True)
    outr[0] = _f32dot(a, vr[0])


def _attn(qkv3, n):
    return pl.pallas_call(
        functools.partial(_attn_body, nsrc=n),
        grid=(TF_HEADS, n // RB),
        in_specs=[
            pl.BlockSpec((1, RB, HEAD_DIM), lambda h, i: (h, i, 0)),
            pl.BlockSpec((1, n, HEAD_DIM), lambda h, i: (TF_HEADS + h, 0, 0)),
            pl.BlockSpec((1, n, HEAD_DIM), lambda h, i: (2 * TF_HEADS + h, 0, 0)),
        ],
        out_specs=pl.BlockSpec((1, RB, HEAD_DIM), lambda h, i: (h, i, 0)),
        out_shape=jax.ShapeDtypeStruct((TF_HEADS, n, HEAD_DIM), jnp.float32),
    )(qkv3, qkv3, qkv3)


def _ln(x, gg, bb):
    mu = jnp.mean(x, axis=-1, keepdims=True)
    var = jnp.mean((x - mu) ** 2, axis=-1, keepdims=True)
    return (x - mu) / jnp.sqrt(var + 1e-5) * gg + bb


def _post_common(orf, xr, woT, ob, l1g, l1b, w1, b1, w2, b2, l2g, l2b):
    o = jnp.concatenate([orf[h] for h in range(TF_HEADS)], axis=1)
    y = _f32dot(o, woT[...]) + ob[...] + xr[...]
    y = _ln(y, l1g[...], l1b[...])
    hh = _f32dot(y, w1[...]) + b1[...]
    hh = jnp.where(hh >= 0.0, hh, 0.2 * hh)
    hh = _f32dot(hh, w2[...]) + b2[...]
    return _ln(y + hh, l2g[...], l2b[...])


def _post_body(orf, xr, woT, ob, l1g, l1b, w1, b1, w2, b2, l2g, l2b, outr):
    outr[...] = _post_common(orf, xr, woT, ob, l1g, l1b, w1, b1, w2, b2, l2g, l2b)


def _post_fc_body(orf, xr, woT, ob, l1g, l1b, w1, b1, w2, b2, l2g, l2b,
                  fcw, fcb, outr, fusedr):
    out = _post_common(orf, xr, woT, ob, l1g, l1b, w1, b1, w2, b2, l2g, l2b)
    outr[...] = out
    fusedr[...] = _f32dot(out, fcw[...]) + fcb[...]


def _row_spec(n, d):
    return pl.BlockSpec((RB, d), lambda i: (i, 0))


def _const_spec(shape):
    return pl.BlockSpec(shape, lambda i: (0, 0))


def _post(o3, x, woT, ob, l1g, l1b, w1, b1, w2, b2, l2g, l2b, fcw=None, fcb=None):
    n = x.shape[0]
    ospec = pl.BlockSpec((TF_HEADS, RB, HEAD_DIM), lambda i: (0, i, 0))
    wspecs = [
        _const_spec((D_MODEL, D_MODEL)), _const_spec((1, D_MODEL)),
        _const_spec((1, D_MODEL)), _const_spec((1, D_MODEL)),
        _const_spec((D_MODEL, D_MODEL)), _const_spec((1, D_MODEL)),
        _const_spec((D_MODEL, D_MODEL)), _const_spec((1, D_MODEL)),
        _const_spec((1, D_MODEL)), _const_spec((1, D_MODEL)),
    ]
    if fcw is None:
        return pl.pallas_call(
            _post_body,
            grid=(n // RB,),
            in_specs=[ospec, _row_spec(n, D_MODEL)] + wspecs,
            out_specs=_row_spec(n, D_MODEL),
            out_shape=jax.ShapeDtypeStruct((n, D_MODEL), jnp.float32),
        )(o3, x, woT, ob, l1g, l1b, w1, b1, w2, b2, l2g, l2b)
    return pl.pallas_call(
        _post_fc_body,
        grid=(n // RB,),
        in_specs=[ospec, _row_spec(n, D_MODEL)] + wspecs
        + [_const_spec((D_MODEL, FUSED)), _const_spec((1, FUSED))],
        out_specs=[_row_spec(n, D_MODEL), _row_spec(n, FUSED)],
        out_shape=[jax.ShapeDtypeStruct((n, D_MODEL), jnp.float32),
                   jax.ShapeDtypeStruct((n, FUSED), jnp.float32)],
    )(o3, x, woT, ob, l1g, l1b, w1, b1, w2, b2, l2g, l2b, fcw, fcb)


def _elo_body(xr, fr, wsg, wsf, wdg, wdf, wu, elr, err, usr, vdr):
    x = xr[...]
    gpart = x[:, :G_DIM]
    ppart = x[:, G_DIM:]
    f = fr[...]
    fs = _f32dot(f, wsf[...])
    fd = _f32dot(f, wdf[...])
    el_g = _f32dot(gpart, wsg[...]) + fs
    el_p = _f32dot(ppart, wsg[...]) + fs
    er_g = _f32dot(gpart, wdg[...]) + fd
    er_p = _f32dot(ppart, wdg[...]) + fd
    el = jnp.concatenate([el_g, el_p], axis=1)
    er = jnp.concatenate([er_g, er_p], axis=1)
    elr[...] = el
    err[...] = er
    usr[...] = _f32dot(el, wu[...])
    vdr[...] = _f32dot(er, wu[...])


def _elo(x, fused, wsg, wsf, wdg, wdf, wu):
    n = x.shape[0]
    hw = GAT_HEADS * G_DIM
    return pl.pallas_call(
        _elo_body,
        grid=(n // RB,),
        in_specs=[
            _row_spec(n, D_MODEL), _row_spec(n, FUSED),
            _const_spec((G_DIM, hw)), _const_spec((FUSED, hw)),
            _const_spec((G_DIM, hw)), _const_spec((FUSED, hw)),
            _const_spec((2 * hw, LANES)),
        ],
        out_specs=[pl.BlockSpec((RB, 2 * hw), lambda i: (i, 0)),
                   pl.BlockSpec((RB, 2 * hw), lambda i: (i, 0)),
                   pl.BlockSpec((RB, LANES), lambda i: (i, 0)),
                   pl.BlockSpec((RB, LANES), lambda i: (i, 0))],
        out_shape=[jax.ShapeDtypeStruct((n, 2 * hw), jnp.float32),
                   jax.ShapeDtypeStruct((n, 2 * hw), jnp.float32),
                   jax.ShapeDtypeStruct((n, LANES), jnp.float32),
                   jax.ShapeDtypeStruct((n, LANES), jnp.float32)],
    )(x, fused, wsg, wsf, wdg, wdf, wu)


def _gat_w_sc(el, er, us, vd, src, dst, attn_flat, n_src, n_dst, E):
    e_per_w = E // NW
    C = 32
    n_chunks = e_per_w // C
    mesh = plsc.VectorSubcoreMesh(core_axis_name="c", subcore_axis_name="s")

    @functools.partial(
        pl.kernel,
        out_type=jax.ShapeDtypeStruct((4, E), jnp.float32),
        mesh=mesh,
        compiler_params=pltpu.CompilerParams(needs_layout_passes=False),
        scratch_types=[
            pltpu.VMEM((C,), jnp.int32),
            pltpu.VMEM((C,), jnp.int32),
            pltpu.VMEM((C, 2 * GAT_HEADS * G_DIM), jnp.float32),
            pltpu.VMEM((C, 2 * GAT_HEADS * G_DIM), jnp.float32),
            pltpu.VMEM((n_src * LANES,), jnp.float32),
            pltpu.VMEM((n_dst * LANES,), jnp.float32),
            pltpu.VMEM((C, LANES), jnp.float32),
            pltpu.VMEM((4, C), jnp.float32),
            pltpu.VMEM((GAT_HEADS * G_DIM,), jnp.float32),
            pltpu.SemaphoreType.DMA,
            pltpu.SemaphoreType.DMA,
        ],
    )
    def w_kernel(el_hbm, er_hbm, us_hbm, vd_hbm, src_hbm, dst_hbm, attn_hbm,
                 out_hbm, src_v, dst_v, el_v, er_v, us_tab, vd_tab, w16_v,
                 wc_v, attn_v, sem1, sem2):
        cid = lax.axis_index("c")
        sid = lax.axis_index("s")
        wid = sid * NC + cid
        zv = jnp.zeros((LANES,), jnp.float32)
        io = lax.iota(jnp.int32, LANES)
        pltpu.sync_copy(attn_hbm, attn_v)
        pltpu.sync_copy(us_hbm, us_tab)
        pltpu.sync_copy(vd_hbm.at[pl.ds(0, n_dst * LANES)], vd_tab)

        attn_c = [attn_v[pl.ds(j * LANES, LANES)] for j in range(16)]
        perms = [jnp.bitwise_xor(io, s) for s in (8, 4, 2, 1)]

        def lanesum(v):
            for pm in perms:
                v = v + jnp.take(v, pm)
            return v

        base = wid * e_per_w

        def chunk(ci, _):
            b = base + ci * C
            pltpu.sync_copy(src_hbm.at[pl.ds(b, C)], src_v)
            pltpu.sync_copy(dst_hbm.at[pl.ds(b, C)], dst_v)
            cp1 = pltpu.async_copy(el_hbm.at[src_v], el_v, sem1)
            cp2 = pltpu.async_copy(er_hbm.at[dst_v], er_v, sem2)
            cp1.wait()
            cp2.wait()

            def edge(e, _):
                rv = []
                for m in range(2):
                    for h in range(GAT_HEADS):
                        acc = zv
                        for k in range(8):
                            j = h * 8 + k
                            v = (el_v[e, pl.ds(m * 256 + j * LANES, LANES)]
                                 + er_v[e, pl.ds(m * 256 + j * LANES, LANES)])
                            acc = acc + jnp.abs(v) * attn_c[j]
                        rv.append(lanesum(acc))
                w16_v[e] = jnp.where(io == 0, rv[0],
                                     jnp.where(io == 1, rv[1],
                                               jnp.where(io == 2, rv[2],
                                                         jnp.where(io == 3, rv[3],
                                                                   zv))))
                return 0

            lax.fori_loop(0, C, edge, 0)
            for g in range(C // LANES):
                ridx = g * LANES + io
                sidx = src_v[pl.ds(g * LANES, LANES)]
                didx = dst_v[pl.ds(g * LANES, LANES)]
                for comp in range(4):
                    compv = jnp.full((LANES,), comp, jnp.int32)
                    col = plsc.load_gather(w16_v, [ridx, compv])
                    uvals = plsc.load_gather(us_tab, [sidx * LANES + comp])
                    vvals = plsc.load_gather(vd_tab, [didx * LANES + comp])
                    wc_v[comp, pl.ds(g * LANES, LANES)] = jnp.exp(
                        0.6 * (uvals + vvals) + 0.4 * col)
            for comp in range(4):
                pltpu.sync_copy(wc_v.at[comp], out_hbm.at[comp, pl.ds(b, C)])
            return 0

        lax.fori_loop(0, n_chunks, chunk, 0)

    return w_kernel(el, er, us, vd, src, dst, attn_flat)


def _gat_A_sc(src, dst, w4, n_src, n_dst, E):
    rpt2 = n_dst // NW
    CH = 2048
    n_chunks = E // CH
    mesh = plsc.VectorSubcoreMesh(core_axis_name="c", subcore_axis_name="s")

    @functools.partial(
        pl.kernel,
        out_type=jax.ShapeDtypeStruct((4, n_dst, n_src), jnp.float32),
        mesh=mesh,
        compiler_params=pltpu.CompilerParams(needs_layout_passes=False),
        scratch_types=[
            pltpu.VMEM((CH,), jnp.int32),
            pltpu.VMEM((CH,), jnp.int32),
            pltpu.VMEM((CH,), jnp.float32),
            pltpu.VMEM((rpt2, n_src), jnp.float32),
        ],
    )
    def a_kernel(src_hbm, dst_hbm, w_hbm, out_hbm, sidx_v, didx_v, wv_v, A_v):
        cid = lax.axis_index("c")
        sid = lax.axis_index("s")
        wid = sid * NC + cid
        lo = wid * rpt2
        io = lax.iota(jnp.int32, LANES)
        zv = jnp.zeros((LANES,), jnp.float32)

        for comp in range(4):
            def zrow(r, _):
                for kk in range(n_src // LANES):
                    A_v[r, pl.ds(kk * LANES, LANES)] = zv
                return 0

            lax.fori_loop(0, rpt2, zrow, 0)

            def chunk(ci, _):
                b = ci * CH
                pltpu.sync_copy(src_hbm.at[pl.ds(b, CH)], sidx_v)
                pltpu.sync_copy(dst_hbm.at[pl.ds(b, CH)], didx_v)
                pltpu.sync_copy(w_hbm.at[comp, pl.ds(b, CH)], wv_v)

                def grp(g, _):
                    s = sidx_v[pl.ds(g * LANES, LANES)]
                    d = didx_v[pl.ds(g * LANES, LANES)]
                    w = wv_v[pl.ds(g * LANES, LANES)]
                    rel = d - lo
                    msk = (rel >= 0) & (rel < rpt2)
                    plsc.addupdate_scatter(A_v, [rel, s], w, mask=msk)
                    return 0

                lax.fori_loop(0, CH // LANES, grp, 0)
                return 0

            lax.fori_loop(0, n_chunks, chunk, 0)
            pltpu.sync_copy(A_v, out_hbm.at[comp, pl.ds(lo, rpt2)])

    return a_kernel(src, dst, w4)


RBD = 256


def _gatA_body(ar, elr, fcw, fcb, gb, outr):
    outs = []
    for m in range(2):
        cols = []
        for h in range(GAT_HEADS):
            c = 2 * m + h
            Ac = ar[c]
            num = _f32dot(Ac, elr[:, c * G_DIM:(c + 1) * G_DIM])
            den = jnp.sum(Ac, axis=1, keepdims=True)
            cols.append(jnp.where(den > 0.0, num / den, 0.0))
        rst = jnp.concatenate(cols, axis=1) + gb[...]
        outs.append(_f32dot(rst, fcw[...]) + fcb[...])
    outr[...] = jnp.concatenate(outs, axis=1)


def _gatA_comb(A, el, fcw, fcb, gbias, n_dst, n_src):
    return pl.pallas_call(
        _gatA_body,
        grid=(n_dst // RBD,),
        in_specs=[
            pl.BlockSpec((4, RBD, n_src), lambda i: (0, i, 0)),
            pl.BlockSpec((n_src, 2 * GAT_HEADS * G_DIM), lambda i: (0, 0)),
            pl.BlockSpec((GAT_HEADS * G_DIM, G_DIM), lambda i: (0, 0)),
            pl.BlockSpec((1, G_DIM), lambda i: (0, 0)),
            pl.BlockSpec((1, GAT_HEADS * G_DIM), lambda i: (0, 0)),
        ],
        out_specs=pl.BlockSpec((RBD, D_MODEL), lambda i: (i, 0)),
        out_shape=jax.ShapeDtypeStruct((n_dst, D_MODEL), jnp.float32),
    )(A, el, fcw, fcb, gbias)


def _up_body(xr, wr, br, outr):
    outr[...] = _f32dot(xr[:, G_DIM:], wr[...]) + br[...]


def _up(x, w, b):
    return pl.pallas_call(
        _up_body,
        out_shape=jax.ShapeDtypeStruct((N2, P_DIM), jnp.float32),
    )(x, w, b)


def kernel(g, p, src0, dst0, src1, dst1, params):
    P = params
    p_flat = p.reshape(N0, P_DIM)
    nh = 3 * TF_HEADS
    in_w3 = P['in_proj_w'].T.reshape(D_MODEL, nh, HEAD_DIM).transpose(1, 0, 2)
    in_b3 = P['in_proj_b'].reshape(nh, 1, HEAD_DIM)
    wo_T = P['out_proj_w'].T
    ob = P['out_proj_b'].reshape(1, -1)
    l1g = P['ln1_g'].reshape(1, -1)
    l1b = P['ln1_b'].reshape(1, -1)
    l2g = P['ln2_g'].reshape(1, -1)
    l2b = P['ln2_b'].reshape(1, -1)
    b1 = P['ffn_b1'].reshape(1, -1)
    b2 = P['ffn_b2'].reshape(1, -1)
    fcb_e = P['enc_fc_b'].reshape(1, -1)
    ws_g = P['gat_w_src'][:G_DIM]
    ws_f = P['gat_w_src'][G_DIM:]
    wd_g = P['gat_w_dst'][:G_DIM]
    wd_f = P['gat_w_dst'][G_DIM:]
    attn_flat = P['gat_attn'].reshape(-1)
    eye4 = jnp.eye(4, LANES, dtype=jnp.float32)
    blocks = jnp.stack([P['gat_attn'][h] for _m in range(2)
                        for h in range(GAT_HEADS)])
    wu = (blocks[:, :, None] * eye4[:, None, :]).reshape(4 * G_DIM, LANES)
    gfc_b = P['gat_fc_b'].reshape(1, -1)
    gbias = P['gat_bias'].reshape(1, -1)

    x = _down(g, p_flat, P['p_down_w'], P['p_down_b'].reshape(1, -1),
              P['g_mask_token'], P['p_mask_token'])

    for (src, dst, n_src, n_dst, E) in [(src0, dst0, N0, N1, E0),
                                        (src1, dst1, N1, N2, E1)]:
        h = x
        fused = None
        for layer in range(2):
            qkv = _qkv(h, in_w3, in_b3)
            o = _attn(qkv, n_src)
            if layer == 0:
                h = _post(o, h, wo_T, ob, l1g, l1b, P['ffn_w1'], b1,
                          P['ffn_w2'], b2, l2g, l2b)
            else:
                h, fused = _post(o, h, wo_T, ob, l1g, l1b, P['ffn_w1'], b1,
                                 P['ffn_w2'], b2, l2g, l2b,
                                 fcw=P['enc_fc_w'], fcb=fcb_e)
        el, er, us, vd = _elo(x, fused, ws_g, ws_f, wd_g, wd_f, wu)
        w4 = _gat_w_sc(el, er, us.reshape(-1), vd.reshape(-1), src, dst,
                       attn_flat, n_src, n_dst, E)
        A = _gat_A_sc(src, dst, w4, n_src, n_dst, E)
        x = _gatA_comb(A, el, P['gat_fc_w'], gfc_b, gbias, n_dst, n_src)

    p_out = _up(x, P['p_up_w'], P['p_up_b'].reshape(1, -1))
    return x[:, :G_DIM], p_out.reshape(N2, EMB_CHAN, PATCH, PATCH)


# speedup vs baseline: 1.1444x; 1.1444x over previous
import functools

import jax
import jax.numpy as jnp
from jax import lax
from jax.experimental import pallas as pl
from jax.experimental.pallas import tpu as pltpu
from jax.experimental.pallas import tpu_sc as plsc

N0, N1, N2 = 2048, 1024, 512
G_DIM = 128
EMB_CHAN = 16
PATCH = 16
P_DIM = EMB_CHAN * PATCH * PATCH
FUSED = 16
D_MODEL = 2 * G_DIM
TF_HEADS = 4
HEAD_DIM = D_MODEL // TF_HEADS
GAT_HEADS = 2
E0, E1 = 32768, 16384
NEG = -1e30

RB = 256

NC, NS, LANES = 2, 16, 16
NW = NC * NS


def _f32dot(a, b):
    return jnp.dot(a, b, preferred_element_type=jnp.float32)


def _down_body(gr, pr, wr, br, gt, pt, outr):
    i = pl.program_id(0)
    pp = _f32dot(pr[...], wr[...]) + br[...]
    gg = gr[...]
    masked = i < (N2 // RB)
    gg = jnp.where(masked, jnp.broadcast_to(gt[...], gg.shape), gg)
    pp = jnp.where(masked, jnp.broadcast_to(pt[...], pp.shape), pp)
    outr[...] = jnp.concatenate([gg, pp], axis=1)


def _down(g, p_flat, w, b, gtok, ptok):
    return pl.pallas_call(
        _down_body,
        grid=(N0 // RB,),
        in_specs=[
            pl.BlockSpec((RB, G_DIM), lambda i: (i, 0)),
            pl.BlockSpec((RB, P_DIM), lambda i: (i, 0)),
            pl.BlockSpec((P_DIM, G_DIM), lambda i: (0, 0)),
            pl.BlockSpec((1, G_DIM), lambda i: (0, 0)),
            pl.BlockSpec((1, G_DIM), lambda i: (0, 0)),
            pl.BlockSpec((1, G_DIM), lambda i: (0, 0)),
        ],
        out_specs=pl.BlockSpec((RB, D_MODEL), lambda i: (i, 0)),
        out_shape=jax.ShapeDtypeStruct((N0, D_MODEL), jnp.float32),
    )(g, p_flat, w, b, gtok, ptok)


def _qkv_body(xr, wr, br, outr):
    outr[0] = _f32dot(xr[...], wr[0]) + br[0]


def _qkv(x, w3, b3):
    n = x.shape[0]
    nh = 3 * TF_HEADS
    return pl.pallas_call(
        _qkv_body,
        grid=(nh,),
        in_specs=[
            pl.BlockSpec((n, D_MODEL), lambda j: (0, 0)),
            pl.BlockSpec((1, D_MODEL, HEAD_DIM), lambda j: (j, 0, 0)),
            pl.BlockSpec((1, 1, HEAD_DIM), lambda j: (j, 0, 0)),
        ],
        out_specs=pl.BlockSpec((1, n, HEAD_DIM), lambda j: (j, 0, 0)),
        out_shape=jax.ShapeDtypeStruct((nh, n, HEAD_DIM), jnp.float32),
    )(x, w3, b3)


def _attn_body(qr, kr, vr, outr, *, nsrc):
    i = pl.program_id(1)
    s = lax.dot_general(qr[0], kr[0], (((1,), (1,)), ((), ())),
                        preferred_element_type=jnp.float32) * (1.0 / 8.0)
    rows = i * RB + lax.broadcasted_iota(jnp.int32, (RB, nsrc), 0)
    cols = lax.broadcasted_iota(jnp.int32, (RB, nsrc), 1)
    s = jnp.where((rows < N2) & (cols < N2), NEG, s)
    m = jnp.max(s, axis=1, keepdims=True)
    e = jnp.exp(s - m)
    a = e / jnp.sum(e, axis=1, keepdims=True)
    outr[0] = _f32dot(a, vr[0])


def _attn(qkv3, n):
    return pl.pallas_call(
        functools.partial(_attn_body, nsrc=n),
        grid=(TF_HEADS, n // RB),
        in_specs=[
            pl.BlockSpec((1, RB, HEAD_DIM), lambda h, i: (h, i, 0)),
            pl.BlockSpec((1, n, HEAD_DIM), lambda h, i: (TF_HEADS + h, 0, 0)),
            pl.BlockSpec((1, n, HEAD_DIM), lambda h, i: (2 * TF_HEADS + h, 0, 0)),
        ],
        out_specs=pl.BlockSpec((1, RB, HEAD_DIM), lambda h, i: (h, i, 0)),
        out_shape=jax.ShapeDtypeStruct((TF_HEADS, n, HEAD_DIM), jnp.float32),
    )(qkv3, qkv3, qkv3)


def _ln(x, gg, bb):
    mu = jnp.mean(x, axis=-1, keepdims=True)
    var = jnp.mean((x - mu) ** 2, axis=-1, keepdims=True)
    return (x - mu) / jnp.sqrt(var + 1e-5) * gg + bb


def _post_common(orf, xr, woT, ob, l1g, l1b, w1, b1, w2, b2, l2g, l2b):
    o = jnp.concatenate([orf[h] for h in range(TF_HEADS)], axis=1)
    y = _f32dot(o, woT[...]) + ob[...] + xr[...]
    y = _ln(y, l1g[...], l1b[...])
    hh = _f32dot(y, w1[...]) + b1[...]
    hh = jnp.where(hh >= 0.0, hh, 0.2 * hh)
    hh = _f32dot(hh, w2[...]) + b2[...]
    return _ln(y + hh, l2g[...], l2b[...])


def _post_body(orf, xr, woT, ob, l1g, l1b, w1, b1, w2, b2, l2g, l2b, outr):
    outr[...] = _post_common(orf, xr, woT, ob, l1g, l1b, w1, b1, w2, b2, l2g, l2b)


def _post_fc_body(orf, xr, woT, ob, l1g, l1b, w1, b1, w2, b2, l2g, l2b,
                  fcw, fcb, outr, fusedr):
    out = _post_common(orf, xr, woT, ob, l1g, l1b, w1, b1, w2, b2, l2g, l2b)
    outr[...] = out
    fusedr[...] = _f32dot(out, fcw[...]) + fcb[...]


def _row_spec(n, d):
    return pl.BlockSpec((RB, d), lambda i: (i, 0))


def _const_spec(shape):
    return pl.BlockSpec(shape, lambda i: (0, 0))


def _post(o3, x, woT, ob, l1g, l1b, w1, b1, w2, b2, l2g, l2b, fcw=None, fcb=None):
    n = x.shape[0]
    ospec = pl.BlockSpec((TF_HEADS, RB, HEAD_DIM), lambda i: (0, i, 0))
    wspecs = [
        _const_spec((D_MODEL, D_MODEL)), _const_spec((1, D_MODEL)),
        _const_spec((1, D_MODEL)), _const_spec((1, D_MODEL)),
        _const_spec((D_MODEL, D_MODEL)), _const_spec((1, D_MODEL)),
        _const_spec((D_MODEL, D_MODEL)), _const_spec((1, D_MODEL)),
        _const_spec((1, D_MODEL)), _const_spec((1, D_MODEL)),
    ]
    if fcw is None:
        return pl.pallas_call(
            _post_body,
            grid=(n // RB,),
            in_specs=[ospec, _row_spec(n, D_MODEL)] + wspecs,
            out_specs=_row_spec(n, D_MODEL),
            out_shape=jax.ShapeDtypeStruct((n, D_MODEL), jnp.float32),
        )(o3, x, woT, ob, l1g, l1b, w1, b1, w2, b2, l2g, l2b)
    return pl.pallas_call(
        _post_fc_body,
        grid=(n // RB,),
        in_specs=[ospec, _row_spec(n, D_MODEL)] + wspecs
        + [_const_spec((D_MODEL, FUSED)), _const_spec((1, FUSED))],
        out_specs=[_row_spec(n, D_MODEL), _row_spec(n, FUSED)],
        out_shape=[jax.ShapeDtypeStruct((n, D_MODEL), jnp.float32),
                   jax.ShapeDtypeStruct((n, FUSED), jnp.float32)],
    )(o3, x, woT, ob, l1g, l1b, w1, b1, w2, b2, l2g, l2b, fcw, fcb)


def _elo_body(xr, fr, wsg, wsf, wdg, wdf, wu, elr, err, usr, vdr):
    x = xr[...]
    gpart = x[:, :G_DIM]
    ppart = x[:, G_DIM:]
    f = fr[...]
    fs = _f32dot(f, wsf[...])
    fd = _f32dot(f, wdf[...])
    el_g = _f32dot(gpart, wsg[...]) + fs
    el_p = _f32dot(ppart, wsg[...]) + fs
    er_g = _f32dot(gpart, wdg[...]) + fd
    er_p = _f32dot(ppart, wdg[...]) + fd
    el = jnp.concatenate([el_g, el_p], axis=1)
    er = jnp.concatenate([er_g, er_p], axis=1)
    elr[...] = el
    err[...] = er
    usr[...] = _f32dot(el, wu[...])
    vdr[...] = _f32dot(er, wu[...])


def _elo(x, fused, wsg, wsf, wdg, wdf, wu):
    n = x.shape[0]
    hw = GAT_HEADS * G_DIM
    return pl.pallas_call(
        _elo_body,
        grid=(n // RB,),
        in_specs=[
            _row_spec(n, D_MODEL), _row_spec(n, FUSED),
            _const_spec((G_DIM, hw)), _const_spec((FUSED, hw)),
            _const_spec((G_DIM, hw)), _const_spec((FUSED, hw)),
            _const_spec((2 * hw, LANES)),
        ],
        out_specs=[pl.BlockSpec((RB, 2 * hw), lambda i: (i, 0)),
                   pl.BlockSpec((RB, 2 * hw), lambda i: (i, 0)),
                   pl.BlockSpec((RB, LANES), lambda i: (i, 0)),
                   pl.BlockSpec((RB, LANES), lambda i: (i, 0))],
        out_shape=[jax.ShapeDtypeStruct((n, 2 * hw), jnp.float32),
                   jax.ShapeDtypeStruct((n, 2 * hw), jnp.float32),
                   jax.ShapeDtypeStruct((n, LANES), jnp.float32),
                   jax.ShapeDtypeStruct((n, LANES), jnp.float32)],
    )(x, fused, wsg, wsf, wdg, wdf, wu)


def _gat_w_sc(el, er, us, vd, src, dst, attn_flat, n_src, n_dst, E):
    e_per_w = E // NW
    C = 32
    n_chunks = e_per_w // C
    mesh = plsc.VectorSubcoreMesh(core_axis_name="c", subcore_axis_name="s")

    @functools.partial(
        pl.kernel,
        out_type=jax.ShapeDtypeStruct((4, E), jnp.float32),
        mesh=mesh,
        compiler_params=pltpu.CompilerParams(needs_layout_passes=False),
        scratch_types=[
            pltpu.VMEM((e_per_w,), jnp.int32),
            pltpu.VMEM((e_per_w,), jnp.int32),
            pltpu.VMEM((C, 2 * GAT_HEADS * G_DIM), jnp.float32),
            pltpu.VMEM((C, 2 * GAT_HEADS * G_DIM), jnp.float32),
            pltpu.VMEM((C, 2 * GAT_HEADS * G_DIM), jnp.float32),
            pltpu.VMEM((C, 2 * GAT_HEADS * G_DIM), jnp.float32),
            pltpu.VMEM((n_src * LANES,), jnp.float32),
            pltpu.VMEM((n_dst * LANES,), jnp.float32),
            pltpu.VMEM((C, LANES), jnp.float32),
            pltpu.VMEM((4, C), jnp.float32),
            pltpu.VMEM((GAT_HEADS * G_DIM,), jnp.float32),
            pltpu.SemaphoreType.DMA,
            pltpu.SemaphoreType.DMA,
            pltpu.SemaphoreType.DMA,
            pltpu.SemaphoreType.DMA,
        ],
    )
    def w_kernel(el_hbm, er_hbm, us_hbm, vd_hbm, src_hbm, dst_hbm, attn_hbm,
                 out_hbm, src_v, dst_v, el0_v, er0_v, el1_v, er1_v, us_tab,
                 vd_tab, w16_v, wc_v, attn_v, sa0, sb0, sa1, sb1):
        cid = lax.axis_index("c")
        sid = lax.axis_index("s")
        wid = sid * NC + cid
        zv = jnp.zeros((LANES,), jnp.float32)
        io = lax.iota(jnp.int32, LANES)
        base = wid * e_per_w
        pltpu.sync_copy(attn_hbm, attn_v)
        pltpu.sync_copy(src_hbm.at[pl.ds(base, e_per_w)], src_v)
        pltpu.sync_copy(dst_hbm.at[pl.ds(base, e_per_w)], dst_v)
        pltpu.sync_copy(us_hbm, us_tab)
        pltpu.sync_copy(vd_hbm.at[pl.ds(0, n_dst * LANES)], vd_tab)

        attn_c = [attn_v[pl.ds(j * LANES, LANES)] for j in range(16)]
        perms = [jnp.bitwise_xor(io, s) for s in (8, 4, 2, 1)]
        slots = [(el0_v, er0_v, sa0, sb0), (el1_v, er1_v, sa1, sb1)]

        def lanesum(v):
            for pm in perms:
                v = v + jnp.take(v, pm)
            return v

        def start(ci, slot):
            elv, erv, s1, s2 = slots[slot]
            idx = pl.ds(ci * C, C)
            pltpu.async_copy(el_hbm.at[src_v.at[idx]], elv, s1)
            pltpu.async_copy(er_hbm.at[dst_v.at[idx]], erv, s2)

        def wait(slot):
            elv, erv, s1, s2 = slots[slot]
            idx = pl.ds(0, C)
            pltpu.make_async_copy(el_hbm.at[src_v.at[idx]], elv, s1).wait()
            pltpu.make_async_copy(er_hbm.at[dst_v.at[idx]], erv, s2).wait()

        def compute(ci, slot):
            elv, erv, _, _ = slots[slot]

            def edge(e, _):
                rv = []
                for m in range(2):
                    for h in range(GAT_HEADS):
                        acc = zv
                        for k in range(8):
                            j = h * 8 + k
                            v = (elv[e, pl.ds(m * 256 + j * LANES, LANES)]
                                 + erv[e, pl.ds(m * 256 + j * LANES, LANES)])
                            acc = acc + jnp.abs(v) * attn_c[j]
                        rv.append(lanesum(acc))
                w16_v[e] = jnp.where(io == 0, rv[0],
                                     jnp.where(io == 1, rv[1],
                                               jnp.where(io == 2, rv[2],
                                                         jnp.where(io == 3, rv[3],
                                                                   zv))))
                return 0

            lax.fori_loop(0, C, edge, 0)
            for g in range(C // LANES):
                ridx = g * LANES + io
                sidx = src_v[pl.ds(ci * C + g * LANES, LANES)]
                didx = dst_v[pl.ds(ci * C + g * LANES, LANES)]
                for comp in range(4):
                    compv = jnp.full((LANES,), comp, jnp.int32)
                    col = plsc.load_gather(w16_v, [ridx, compv])
                    uvals = plsc.load_gather(us_tab, [sidx * LANES + comp])
                    vvals = plsc.load_gather(vd_tab, [didx * LANES + comp])
                    wc_v[comp, pl.ds(g * LANES, LANES)] = jnp.exp(
                        0.6 * (uvals + vvals) + 0.4 * col)
            for comp in range(4):
                pltpu.sync_copy(
                    wc_v.at[comp],
                    out_hbm.at[comp, pl.ds(base + ci * C, C)])

        start(0, 0)

        def pair(g, _):
            c0 = 2 * g
            start(c0 + 1, 1)
            wait(0)
            compute(c0, 0)
            start(jnp.minimum(c0 + 2, n_chunks - 2), 0)
            wait(1)
            compute(c0 + 1, 1)
            return 0

        lax.fori_loop(0, n_chunks // 2, pair, 0)
        wait(0)

    return w_kernel(el, er, us, vd, src, dst, attn_flat)


def _gat_A_sc(src, dst, w4, n_src, n_dst, E):
    rpt2 = n_dst // NW
    CH = 2048
    n_chunks = E // CH
    mesh = plsc.VectorSubcoreMesh(core_axis_name="c", subcore_axis_name="s")

    @functools.partial(
        pl.kernel,
        out_type=jax.ShapeDtypeStruct((4, n_dst, n_src), jnp.float32),
        mesh=mesh,
        compiler_params=pltpu.CompilerParams(needs_layout_passes=False),
        scratch_types=[
            pltpu.VMEM((CH,), jnp.int32),
            pltpu.VMEM((CH,), jnp.int32),
            pltpu.VMEM((CH,), jnp.float32),
            pltpu.VMEM((rpt2, n_src), jnp.float32),
        ],
    )
    def a_kernel(src_hbm, dst_hbm, w_hbm, out_hbm, sidx_v, didx_v, wv_v, A_v):
        cid = lax.axis_index("c")
        sid = lax.axis_index("s")
        wid = sid * NC + cid
        lo = wid * rpt2
        io = lax.iota(jnp.int32, LANES)
        zv = jnp.zeros((LANES,), jnp.float32)

        for comp in range(4):
            def zrow(r, _):
                for kk in range(n_src // LANES):
                    A_v[r, pl.ds(kk * LANES, LANES)] = zv
                return 0

            lax.fori_loop(0, rpt2, zrow, 0)

            def chunk(ci, _):
                b = ci * CH
                pltpu.sync_copy(src_hbm.at[pl.ds(b, CH)], sidx_v)
                pltpu.sync_copy(dst_hbm.at[pl.ds(b, CH)], didx_v)
                pltpu.sync_copy(w_hbm.at[comp, pl.ds(b, CH)], wv_v)

                def grp(g, _):
                    s = sidx_v[pl.ds(g * LANES, LANES)]
                    d = didx_v[pl.ds(g * LANES, LANES)]
                    w = wv_v[pl.ds(g * LANES, LANES)]
                    rel = d - lo
                    msk = (rel >= 0) & (rel < rpt2)
                    plsc.addupdate_scatter(A_v, [rel, s], w, mask=msk)
                    return 0

                lax.fori_loop(0, CH // LANES, grp, 0)
                return 0

            lax.fori_loop(0, n_chunks, chunk, 0)
            pltpu.sync_copy(A_v, out_hbm.at[comp, pl.ds(lo, rpt2)])

    return a_kernel(src, dst, w4)


RBD = 256


def _gatA_body(ar, elr, fcw, fcb, gb, outr):
    outs = []
    for m in range(2):
        cols = []
        for h in range(GAT_HEADS):
            c = 2 * m + h
            Ac = ar[c]
            num = _f32dot(Ac, elr[:, c * G_DIM:(c + 1) * G_DIM])
            den = jnp.sum(Ac, axis=1, keepdims=True)
            cols.append(jnp.where(den > 0.0, num / den, 0.0))
        rst = jnp.concatenate(cols, axis=1) + gb[...]
        outs.append(_f32dot(rst, fcw[...]) + fcb[...])
    outr[...] = jnp.concatenate(outs, axis=1)


def _gatA_comb(A, el, fcw, fcb, gbias, n_dst, n_src):
    return pl.pallas_call(
        _gatA_body,
        grid=(n_dst // RBD,),
        in_specs=[
            pl.BlockSpec((4, RBD, n_src), lambda i: (0, i, 0)),
            pl.BlockSpec((n_src, 2 * GAT_HEADS * G_DIM), lambda i: (0, 0)),
            pl.BlockSpec((GAT_HEADS * G_DIM, G_DIM), lambda i: (0, 0)),
            pl.BlockSpec((1, G_DIM), lambda i: (0, 0)),
            pl.BlockSpec((1, GAT_HEADS * G_DIM), lambda i: (0, 0)),
        ],
        out_specs=pl.BlockSpec((RBD, D_MODEL), lambda i: (i, 0)),
        out_shape=jax.ShapeDtypeStruct((n_dst, D_MODEL), jnp.float32),
    )(A, el, fcw, fcb, gbias)


def _up_body(xr, wr, br, outr):
    outr[...] = _f32dot(xr[:, G_DIM:], wr[...]) + br[...]


def _up(x, w, b):
    return pl.pallas_call(
        _up_body,
        out_shape=jax.ShapeDtypeStruct((N2, P_DIM), jnp.float32),
    )(x, w, b)


def kernel(g, p, src0, dst0, src1, dst1, params):
    P = params
    p_flat = p.reshape(N0, P_DIM)
    nh = 3 * TF_HEADS
    in_w3 = P['in_proj_w'].T.reshape(D_MODEL, nh, HEAD_DIM).transpose(1, 0, 2)
    in_b3 = P['in_proj_b'].reshape(nh, 1, HEAD_DIM)
    wo_T = P['out_proj_w'].T
    ob = P['out_proj_b'].reshape(1, -1)
    l1g = P['ln1_g'].reshape(1, -1)
    l1b = P['ln1_b'].reshape(1, -1)
    l2g = P['ln2_g'].reshape(1, -1)
    l2b = P['ln2_b'].reshape(1, -1)
    b1 = P['ffn_b1'].reshape(1, -1)
    b2 = P['ffn_b2'].reshape(1, -1)
    fcb_e = P['enc_fc_b'].reshape(1, -1)
    ws_g = P['gat_w_src'][:G_DIM]
    ws_f = P['gat_w_src'][G_DIM:]
    wd_g = P['gat_w_dst'][:G_DIM]
    wd_f = P['gat_w_dst'][G_DIM:]
    attn_flat = P['gat_attn'].reshape(-1)
    eye4 = jnp.eye(4, LANES, dtype=jnp.float32)
    blocks = jnp.stack([P['gat_attn'][h] for _m in range(2)
                        for h in range(GAT_HEADS)])
    wu = (blocks[:, :, None] * eye4[:, None, :]).reshape(4 * G_DIM, LANES)
    gfc_b = P['gat_fc_b'].reshape(1, -1)
    gbias = P['gat_bias'].reshape(1, -1)

    x = _down(g, p_flat, P['p_down_w'], P['p_down_b'].reshape(1, -1),
              P['g_mask_token'], P['p_mask_token'])

    for (src, dst, n_src, n_dst, E) in [(src0, dst0, N0, N1, E0),
                                        (src1, dst1, N1, N2, E1)]:
        h = x
        fused = None
        for layer in range(2):
            qkv = _qkv(h, in_w3, in_b3)
            o = _attn(qkv, n_src)
            if layer == 0:
                h = _post(o, h, wo_T, ob, l1g, l1b, P['ffn_w1'], b1,
                          P['ffn_w2'], b2, l2g, l2b)
            else:
                h, fused = _post(o, h, wo_T, ob, l1g, l1b, P['ffn_w1'], b1,
                                 P['ffn_w2'], b2, l2g, l2b,
                                 fcw=P['enc_fc_w'], fcb=fcb_e)
        el, er, us, vd = _elo(x, fused, ws_g, ws_f, wd_g, wd_f, wu)
        w4 = _gat_w_sc(el, er, us.reshape(-1), vd.reshape(-1), src, dst,
                       attn_flat, n_src, n_dst, E)
        A = _gat_A_sc(src, dst, w4, n_src, n_dst, E)
        x = _gatA_comb(A, el, P['gat_fc_w'], gfc_b, gbias, n_dst, n_src)

    p_out = _up(x, P['p_up_w'], P['p_up_b'].reshape(1, -1))
    return x[:, :G_DIM], p_out.reshape(N2, EMB_CHAN, PATCH, PATCH)


# speedup vs baseline: 1.4231x; 1.2435x over previous
import functools

import jax
import jax.numpy as jnp
from jax import lax
from jax.experimental import pallas as pl
from jax.experimental.pallas import tpu as pltpu
from jax.experimental.pallas import tpu_sc as plsc

N0, N1, N2 = 2048, 1024, 512
G_DIM = 128
EMB_CHAN = 16
PATCH = 16
P_DIM = EMB_CHAN * PATCH * PATCH
FUSED = 16
D_MODEL = 2 * G_DIM
TF_HEADS = 4
HEAD_DIM = D_MODEL // TF_HEADS
GAT_HEADS = 2
E0, E1 = 32768, 16384
NEG = -1e30

RB = 256

NC, NS, LANES = 2, 16, 16
NW = NC * NS


def _f32dot(a, b):
    return jnp.dot(a, b, preferred_element_type=jnp.float32)


def _down_body(gr, pr, wr, br, gt, pt, outr):
    i = pl.program_id(0)
    pp = _f32dot(pr[...], wr[...]) + br[...]
    gg = gr[...]
    masked = i < (N2 // RB)
    gg = jnp.where(masked, jnp.broadcast_to(gt[...], gg.shape), gg)
    pp = jnp.where(masked, jnp.broadcast_to(pt[...], pp.shape), pp)
    outr[...] = jnp.concatenate([gg, pp], axis=1)


def _down(g, p_flat, w, b, gtok, ptok):
    return pl.pallas_call(
        _down_body,
        grid=(N0 // RB,),
        in_specs=[
            pl.BlockSpec((RB, G_DIM), lambda i: (i, 0)),
            pl.BlockSpec((RB, P_DIM), lambda i: (i, 0)),
            pl.BlockSpec((P_DIM, G_DIM), lambda i: (0, 0)),
            pl.BlockSpec((1, G_DIM), lambda i: (0, 0)),
            pl.BlockSpec((1, G_DIM), lambda i: (0, 0)),
            pl.BlockSpec((1, G_DIM), lambda i: (0, 0)),
        ],
        out_specs=pl.BlockSpec((RB, D_MODEL), lambda i: (i, 0)),
        out_shape=jax.ShapeDtypeStruct((N0, D_MODEL), jnp.float32),
    )(g, p_flat, w, b, gtok, ptok)


def _qkv_body(xr, wr, br, outr):
    outr[0] = _f32dot(xr[...], wr[0]) + br[0]


def _qkv(x, w3, b3):
    n = x.shape[0]
    nh = 3 * TF_HEADS
    return pl.pallas_call(
        _qkv_body,
        grid=(nh,),
        in_specs=[
            pl.BlockSpec((n, D_MODEL), lambda j: (0, 0)),
            pl.BlockSpec((1, D_MODEL, HEAD_DIM), lambda j: (j, 0, 0)),
            pl.BlockSpec((1, 1, HEAD_DIM), lambda j: (j, 0, 0)),
        ],
        out_specs=pl.BlockSpec((1, n, HEAD_DIM), lambda j: (j, 0, 0)),
        out_shape=jax.ShapeDtypeStruct((nh, n, HEAD_DIM), jnp.float32),
    )(x, w3, b3)


def _attn_body(qr, kr, vr, outr, *, nsrc):
    i = pl.program_id(1)
    s = lax.dot_general(qr[0], kr[0], (((1,), (1,)), ((), ())),
                        preferred_element_type=jnp.float32) * (1.0 / 8.0)
    rows = i * RB + lax.broadcasted_iota(jnp.int32, (RB, nsrc), 0)
    cols = lax.broadcasted_iota(jnp.int32, (RB, nsrc), 1)
    s = jnp.where((rows < N2) & (cols < N2), NEG, s)
    m = jnp.max(s, axis=1, keepdims=True)
    e = jnp.exp(s - m)
    a = e / jnp.sum(e, axis=1, keepdims=True)
    outr[0] = _f32dot(a, vr[0])


def _attn(qkv3, n):
    return pl.pallas_call(
        functools.partial(_attn_body, nsrc=n),
        grid=(TF_HEADS, n // RB),
        in_specs=[
            pl.BlockSpec((1, RB, HEAD_DIM), lambda h, i: (h, i, 0)),
            pl.BlockSpec((1, n, HEAD_DIM), lambda h, i: (TF_HEADS + h, 0, 0)),
            pl.BlockSpec((1, n, HEAD_DIM), lambda h, i: (2 * TF_HEADS + h, 0, 0)),
        ],
        out_specs=pl.BlockSpec((1, RB, HEAD_DIM), lambda h, i: (h, i, 0)),
        out_shape=jax.ShapeDtypeStruct((TF_HEADS, n, HEAD_DIM), jnp.float32),
    )(qkv3, qkv3, qkv3)


def _ln(x, gg, bb):
    mu = jnp.mean(x, axis=-1, keepdims=True)
    var = jnp.mean((x - mu) ** 2, axis=-1, keepdims=True)
    return (x - mu) / jnp.sqrt(var + 1e-5) * gg + bb


def _post_common(orf, xr, woT, ob, l1g, l1b, w1, b1, w2, b2, l2g, l2b):
    o = jnp.concatenate([orf[h] for h in range(TF_HEADS)], axis=1)
    y = _f32dot(o, woT[...]) + ob[...] + xr[...]
    y = _ln(y, l1g[...], l1b[...])
    hh = _f32dot(y, w1[...]) + b1[...]
    hh = jnp.where(hh >= 0.0, hh, 0.2 * hh)
    hh = _f32dot(hh, w2[...]) + b2[...]
    return _ln(y + hh, l2g[...], l2b[...])


def _post_body(orf, xr, woT, ob, l1g, l1b, w1, b1, w2, b2, l2g, l2b, outr):
    outr[...] = _post_common(orf, xr, woT, ob, l1g, l1b, w1, b1, w2, b2, l2g, l2b)


def _post_fc_body(orf, xr, woT, ob, l1g, l1b, w1, b1, w2, b2, l2g, l2b,
                  fcw, fcb, outr, fusedr):
    out = _post_common(orf, xr, woT, ob, l1g, l1b, w1, b1, w2, b2, l2g, l2b)
    outr[...] = out
    fusedr[...] = _f32dot(out, fcw[...]) + fcb[...]


def _row_spec(n, d):
    return pl.BlockSpec((RB, d), lambda i: (i, 0))


def _const_spec(shape):
    return pl.BlockSpec(shape, lambda i: (0, 0))


def _post(o3, x, woT, ob, l1g, l1b, w1, b1, w2, b2, l2g, l2b, fcw=None, fcb=None):
    n = x.shape[0]
    ospec = pl.BlockSpec((TF_HEADS, RB, HEAD_DIM), lambda i: (0, i, 0))
    wspecs = [
        _const_spec((D_MODEL, D_MODEL)), _const_spec((1, D_MODEL)),
        _const_spec((1, D_MODEL)), _const_spec((1, D_MODEL)),
        _const_spec((D_MODEL, D_MODEL)), _const_spec((1, D_MODEL)),
        _const_spec((D_MODEL, D_MODEL)), _const_spec((1, D_MODEL)),
        _const_spec((1, D_MODEL)), _const_spec((1, D_MODEL)),
    ]
    if fcw is None:
        return pl.pallas_call(
            _post_body,
            grid=(n // RB,),
            in_specs=[ospec, _row_spec(n, D_MODEL)] + wspecs,
            out_specs=_row_spec(n, D_MODEL),
            out_shape=jax.ShapeDtypeStruct((n, D_MODEL), jnp.float32),
        )(o3, x, woT, ob, l1g, l1b, w1, b1, w2, b2, l2g, l2b)
    return pl.pallas_call(
        _post_fc_body,
        grid=(n // RB,),
        in_specs=[ospec, _row_spec(n, D_MODEL)] + wspecs
        + [_const_spec((D_MODEL, FUSED)), _const_spec((1, FUSED))],
        out_specs=[_row_spec(n, D_MODEL), _row_spec(n, FUSED)],
        out_shape=[jax.ShapeDtypeStruct((n, D_MODEL), jnp.float32),
                   jax.ShapeDtypeStruct((n, FUSED), jnp.float32)],
    )(o3, x, woT, ob, l1g, l1b, w1, b1, w2, b2, l2g, l2b, fcw, fcb)


def _elo_body(xr, fr, wsg, wsf, wdg, wdf, wu, elr, err, usr, vdr):
    x = xr[...]
    gpart = x[:, :G_DIM]
    ppart = x[:, G_DIM:]
    f = fr[...]
    fs = _f32dot(f, wsf[...])
    fd = _f32dot(f, wdf[...])
    el_g = _f32dot(gpart, wsg[...]) + fs
    el_p = _f32dot(ppart, wsg[...]) + fs
    er_g = _f32dot(gpart, wdg[...]) + fd
    er_p = _f32dot(ppart, wdg[...]) + fd
    el = jnp.concatenate([el_g, el_p], axis=1)
    er = jnp.concatenate([er_g, er_p], axis=1)
    elr[...] = el
    err[...] = er
    usr[...] = _f32dot(el, wu[...])
    vdr[...] = _f32dot(er, wu[...])


def _elo(x, fused, wsg, wsf, wdg, wdf, wu):
    n = x.shape[0]
    hw = GAT_HEADS * G_DIM
    return pl.pallas_call(
        _elo_body,
        grid=(n // RB,),
        in_specs=[
            _row_spec(n, D_MODEL), _row_spec(n, FUSED),
            _const_spec((G_DIM, hw)), _const_spec((FUSED, hw)),
            _const_spec((G_DIM, hw)), _const_spec((FUSED, hw)),
            _const_spec((2 * hw, LANES)),
        ],
        out_specs=[pl.BlockSpec((RB, 2 * hw), lambda i: (i, 0)),
                   pl.BlockSpec((RB, 2 * hw), lambda i: (i, 0)),
                   pl.BlockSpec((RB, LANES), lambda i: (i, 0)),
                   pl.BlockSpec((RB, LANES), lambda i: (i, 0))],
        out_shape=[jax.ShapeDtypeStruct((n, 2 * hw), jnp.float32),
                   jax.ShapeDtypeStruct((n, 2 * hw), jnp.float32),
                   jax.ShapeDtypeStruct((n, LANES), jnp.float32),
                   jax.ShapeDtypeStruct((n, LANES), jnp.float32)],
    )(x, fused, wsg, wsf, wdg, wdf, wu)


def _gat_w_sc(el, er, us, vd, src, dst, attn_flat, n_src, n_dst, E):
    e_per_w = E // NW
    C = 32
    n_chunks = e_per_w // C
    mesh = plsc.VectorSubcoreMesh(core_axis_name="c", subcore_axis_name="s")

    @functools.partial(
        pl.kernel,
        out_type=jax.ShapeDtypeStruct((4, E), jnp.float32),
        mesh=mesh,
        compiler_params=pltpu.CompilerParams(needs_layout_passes=False),
        scratch_types=[
            pltpu.VMEM((e_per_w,), jnp.int32),
            pltpu.VMEM((e_per_w,), jnp.int32),
            pltpu.VMEM((C, 2 * GAT_HEADS * G_DIM), jnp.float32),
            pltpu.VMEM((C, 2 * GAT_HEADS * G_DIM), jnp.float32),
            pltpu.VMEM((C, 2 * GAT_HEADS * G_DIM), jnp.float32),
            pltpu.VMEM((C, 2 * GAT_HEADS * G_DIM), jnp.float32),
            pltpu.VMEM((n_src * LANES,), jnp.float32),
            pltpu.VMEM((n_dst * LANES,), jnp.float32),
            pltpu.VMEM((C, LANES), jnp.float32),
            pltpu.VMEM((4, C), jnp.float32),
            pltpu.VMEM((GAT_HEADS * G_DIM,), jnp.float32),
            pltpu.SemaphoreType.DMA,
            pltpu.SemaphoreType.DMA,
            pltpu.SemaphoreType.DMA,
            pltpu.SemaphoreType.DMA,
        ],
    )
    def w_kernel(el_hbm, er_hbm, us_hbm, vd_hbm, src_hbm, dst_hbm, attn_hbm,
                 out_hbm, src_v, dst_v, el0_v, er0_v, el1_v, er1_v, us_tab,
                 vd_tab, w16_v, wc_v, attn_v, sa0, sb0, sa1, sb1):
        cid = lax.axis_index("c")
        sid = lax.axis_index("s")
        wid = sid * NC + cid
        zv = jnp.zeros((LANES,), jnp.float32)
        io = lax.iota(jnp.int32, LANES)
        base = wid * e_per_w
        pltpu.sync_copy(attn_hbm, attn_v)
        pltpu.sync_copy(src_hbm.at[pl.ds(base, e_per_w)], src_v)
        pltpu.sync_copy(dst_hbm.at[pl.ds(base, e_per_w)], dst_v)
        pltpu.sync_copy(us_hbm, us_tab)
        pltpu.sync_copy(vd_hbm.at[pl.ds(0, n_dst * LANES)], vd_tab)

        attn_c = [attn_v[pl.ds(j * LANES, LANES)] for j in range(16)]
        perms = [jnp.bitwise_xor(io, s) for s in (8, 4, 2, 1)]
        slots = [(el0_v, er0_v, sa0, sb0), (el1_v, er1_v, sa1, sb1)]

        def lanesum(v):
            for pm in perms:
                v = v + jnp.take(v, pm)
            return v

        def start(ci, slot):
            elv, erv, s1, s2 = slots[slot]
            idx = pl.ds(ci * C, C)
            pltpu.async_copy(el_hbm.at[src_v.at[idx]], elv, s1)
            pltpu.async_copy(er_hbm.at[dst_v.at[idx]], erv, s2)

        def wait(slot):
            elv, erv, s1, s2 = slots[slot]
            idx = pl.ds(0, C)
            pltpu.make_async_copy(el_hbm.at[src_v.at[idx]], elv, s1).wait()
            pltpu.make_async_copy(er_hbm.at[dst_v.at[idx]], erv, s2).wait()

        def compute(ci, slot):
            elv, erv, _, _ = slots[slot]

            def edge(e, _):
                rv = []
                for m in range(2):
                    for h in range(GAT_HEADS):
                        acc = zv
                        for k in range(8):
                            j = h * 8 + k
                            v = (elv[e, pl.ds(m * 256 + j * LANES, LANES)]
                                 + erv[e, pl.ds(m * 256 + j * LANES, LANES)])
                            acc = acc + jnp.abs(v) * attn_c[j]
                        rv.append(lanesum(acc))
                w16_v[e] = jnp.where(io == 0, rv[0],
                                     jnp.where(io == 1, rv[1],
                                               jnp.where(io == 2, rv[2],
                                                         jnp.where(io == 3, rv[3],
                                                                   zv))))
                return 0

            lax.fori_loop(0, C, edge, 0)
            for g in range(C // LANES):
                ridx = g * LANES + io
                sidx = src_v[pl.ds(ci * C + g * LANES, LANES)]
                didx = dst_v[pl.ds(ci * C + g * LANES, LANES)]
                for comp in range(4):
                    compv = jnp.full((LANES,), comp, jnp.int32)
                    col = plsc.load_gather(w16_v, [ridx, compv])
                    uvals = plsc.load_gather(us_tab, [sidx * LANES + comp])
                    vvals = plsc.load_gather(vd_tab, [didx * LANES + comp])
                    wc_v[comp, pl.ds(g * LANES, LANES)] = jnp.exp(
                        0.6 * (uvals + vvals) + 0.4 * col)
            for comp in range(4):
                pltpu.sync_copy(
                    wc_v.at[comp],
                    out_hbm.at[comp, pl.ds(base + ci * C, C)])

        start(0, 0)

        def pair(g, _):
            c0 = 2 * g
            start(c0 + 1, 1)
            wait(0)
            compute(c0, 0)
            start(jnp.minimum(c0 + 2, n_chunks - 2), 0)
            wait(1)
            compute(c0 + 1, 1)
            return 0

        lax.fori_loop(0, n_chunks // 2, pair, 0)
        wait(0)

    return w_kernel(el, er, us, vd, src, dst, attn_flat)


def _gat_A_sc(src, dst, w4, n_src, n_dst, E):
    rpt2 = n_dst // NW
    CH = 2048
    n_chunks = E // CH
    mesh = plsc.VectorSubcoreMesh(core_axis_name="c", subcore_axis_name="s")

    @functools.partial(
        pl.kernel,
        out_type=jax.ShapeDtypeStruct((4, n_dst, n_src), jnp.float32),
        mesh=mesh,
        compiler_params=pltpu.CompilerParams(needs_layout_passes=False),
        scratch_types=[
            pltpu.VMEM((CH,), jnp.int32),
            pltpu.VMEM((CH,), jnp.int32),
            pltpu.VMEM((CH,), jnp.float32),
            pltpu.VMEM((CH,), jnp.int32),
            pltpu.VMEM((CH,), jnp.int32),
            pltpu.VMEM((CH,), jnp.float32),
            pltpu.VMEM((rpt2, n_src), jnp.float32),
            pltpu.SemaphoreType.DMA,
            pltpu.SemaphoreType.DMA,
            pltpu.SemaphoreType.DMA,
            pltpu.SemaphoreType.DMA,
            pltpu.SemaphoreType.DMA,
            pltpu.SemaphoreType.DMA,
        ],
    )
    def a_kernel(src_hbm, dst_hbm, w_hbm, out_hbm, si0, di0, wv0, si1, di1,
                 wv1, A_v, m0, m1, m2, m3, m4, m5):
        cid = lax.axis_index("c")
        sid = lax.axis_index("s")
        wid = sid * NC + cid
        lo = wid * rpt2
        zv = jnp.zeros((LANES,), jnp.float32)
        slots = [(si0, di0, wv0, m0, m1, m2), (si1, di1, wv1, m3, m4, m5)]

        def start(comp, ci, slot):
            si, di, wv, s1, s2, s3 = slots[slot]
            b = ci * CH
            pltpu.async_copy(src_hbm.at[pl.ds(b, CH)], si, s1)
            pltpu.async_copy(dst_hbm.at[pl.ds(b, CH)], di, s2)
            pltpu.async_copy(w_hbm.at[comp, pl.ds(b, CH)], wv, s3)

        def wfin(slot):
            si, di, wv, s1, s2, s3 = slots[slot]
            pltpu.make_async_copy(src_hbm.at[pl.ds(0, CH)], si, s1).wait()
            pltpu.make_async_copy(dst_hbm.at[pl.ds(0, CH)], di, s2).wait()
            pltpu.make_async_copy(w_hbm.at[0, pl.ds(0, CH)], wv, s3).wait()

        def scat(slot):
            si, di, wv, _, _, _ = slots[slot]

            def grp(g, _):
                s = si[pl.ds(g * LANES, LANES)]
                d = di[pl.ds(g * LANES, LANES)]
                w = wv[pl.ds(g * LANES, LANES)]
                rel = d - lo
                msk = (rel >= 0) & (rel < rpt2)
                plsc.addupdate_scatter(A_v, [rel, s], w, mask=msk)
                return 0

            lax.fori_loop(0, CH // LANES, grp, 0)

        for comp in range(4):
            def zrow(r, _):
                for kk in range(n_src // LANES):
                    A_v[r, pl.ds(kk * LANES, LANES)] = zv
                return 0

            lax.fori_loop(0, rpt2, zrow, 0)

            start(comp, 0, 0)

            def pair(g, _):
                c0 = 2 * g
                start(comp, c0 + 1, 1)
                wfin(0)
                scat(0)
                start(comp, jnp.minimum(c0 + 2, n_chunks - 2), 0)
                wfin(1)
                scat(1)
                return 0

            lax.fori_loop(0, n_chunks // 2, pair, 0)
            wfin(0)
            pltpu.sync_copy(A_v, out_hbm.at[comp, pl.ds(lo, rpt2)])

    return a_kernel(src, dst, w4)


RBD = 256


def _gatA_body(ar, elr, fcw, fcb, gb, outr):
    outs = []
    for m in range(2):
        cols = []
        for h in range(GAT_HEADS):
            c = 2 * m + h
            Ac = ar[c]
            num = _f32dot(Ac, elr[:, c * G_DIM:(c + 1) * G_DIM])
            den = jnp.sum(Ac, axis=1, keepdims=True)
            cols.append(jnp.where(den > 0.0, num / den, 0.0))
        rst = jnp.concatenate(cols, axis=1) + gb[...]
        outs.append(_f32dot(rst, fcw[...]) + fcb[...])
    outr[...] = jnp.concatenate(outs, axis=1)


def _gatA_comb(A, el, fcw, fcb, gbias, n_dst, n_src):
    return pl.pallas_call(
        _gatA_body,
        grid=(n_dst // RBD,),
        in_specs=[
            pl.BlockSpec((4, RBD, n_src), lambda i: (0, i, 0)),
            pl.BlockSpec((n_src, 2 * GAT_HEADS * G_DIM), lambda i: (0, 0)),
            pl.BlockSpec((GAT_HEADS * G_DIM, G_DIM), lambda i: (0, 0)),
            pl.BlockSpec((1, G_DIM), lambda i: (0, 0)),
            pl.BlockSpec((1, GAT_HEADS * G_DIM), lambda i: (0, 0)),
        ],
        out_specs=pl.BlockSpec((RBD, D_MODEL), lambda i: (i, 0)),
        out_shape=jax.ShapeDtypeStruct((n_dst, D_MODEL), jnp.float32),
    )(A, el, fcw, fcb, gbias)


def _up_body(xr, wr, br, outr):
    outr[...] = _f32dot(xr[:, G_DIM:], wr[...]) + br[...]


def _up(x, w, b):
    return pl.pallas_call(
        _up_body,
        out_shape=jax.ShapeDtypeStruct((N2, P_DIM), jnp.float32),
    )(x, w, b)


def kernel(g, p, src0, dst0, src1, dst1, params):
    P = params
    p_flat = p.reshape(N0, P_DIM)
    nh = 3 * TF_HEADS
    in_w3 = P['in_proj_w'].T.reshape(D_MODEL, nh, HEAD_DIM).transpose(1, 0, 2)
    in_b3 = P['in_proj_b'].reshape(nh, 1, HEAD_DIM)
    wo_T = P['out_proj_w'].T
    ob = P['out_proj_b'].reshape(1, -1)
    l1g = P['ln1_g'].reshape(1, -1)
    l1b = P['ln1_b'].reshape(1, -1)
    l2g = P['ln2_g'].reshape(1, -1)
    l2b = P['ln2_b'].reshape(1, -1)
    b1 = P['ffn_b1'].reshape(1, -1)
    b2 = P['ffn_b2'].reshape(1, -1)
    fcb_e = P['enc_fc_b'].reshape(1, -1)
    ws_g = P['gat_w_src'][:G_DIM]
    ws_f = P['gat_w_src'][G_DIM:]
    wd_g = P['gat_w_dst'][:G_DIM]
    wd_f = P['gat_w_dst'][G_DIM:]
    attn_flat = P['gat_attn'].reshape(-1)
    eye4 = jnp.eye(4, LANES, dtype=jnp.float32)
    blocks = jnp.stack([P['gat_attn'][h] for _m in range(2)
                        for h in range(GAT_HEADS)])
    wu = (blocks[:, :, None] * eye4[:, None, :]).reshape(4 * G_DIM, LANES)
    gfc_b = P['gat_fc_b'].reshape(1, -1)
    gbias = P['gat_bias'].reshape(1, -1)

    x = _down(g, p_flat, P['p_down_w'], P['p_down_b'].reshape(1, -1),
              P['g_mask_token'], P['p_mask_token'])

    for (src, dst, n_src, n_dst, E) in [(src0, dst0, N0, N1, E0),
                                        (src1, dst1, N1, N2, E1)]:
        h = x
        fused = None
        for layer in range(2):
            qkv = _qkv(h, in_w3, in_b3)
            o = _attn(qkv, n_src)
            if layer == 0:
                h = _post(o, h, wo_T, ob, l1g, l1b, P['ffn_w1'], b1,
                          P['ffn_w2'], b2, l2g, l2b)
            else:
                h, fused = _post(o, h, wo_T, ob, l1g, l1b, P['ffn_w1'], b1,
                                 P['ffn_w2'], b2, l2g, l2b,
                                 fcw=P['enc_fc_w'], fcb=fcb_e)
        el, er, us, vd = _elo(x, fused, ws_g, ws_f, wd_g, wd_f, wu)
        w4 = _gat_w_sc(el, er, us.reshape(-1), vd.reshape(-1), src, dst,
                       attn_flat, n_src, n_dst, E)
        A = _gat_A_sc(src, dst, w4, n_src, n_dst, E)
        x = _gatA_comb(A, el, P['gat_fc_w'], gfc_b, gbias, n_dst, n_src)

    p_out = _up(x, P['p_up_w'], P['p_up_b'].reshape(1, -1))
    return x[:, :G_DIM], p_out.reshape(N2, EMB_CHAN, PATCH, PATCH)
